# Initial kernel scaffold; baseline (speedup 1.0000x reference)
import jax
import jax.numpy as jnp
from jax.experimental import pallas as pl


def kernel(coords, point_fe, point_batch, face_ds, face_batch):
    raise NotImplementedError("write your pallas kernel here")


# speedup vs baseline: 11.5776x; 11.5776x over previous
import functools

import jax
import jax.numpy as jnp
import numpy as np
from jax import lax
from jax.experimental import pallas as pl
from jax.experimental.pallas import tpu as pltpu
from jax.experimental.pallas import tpu_sc as plsc

NC = 2
NS = 16
NW = NC * NS
CHUNK = 128
NCHUNK = 25
D = 128
DC = 16
THIRD = np.float32(1.0) / np.float32(3.0)


def _sc_body(fe_hbm, co_hbm, g0_hbm, g1_hbm, g2_hbm, dfc_hbm, spt_hbm,
             dpt_hbm, out_fe, out_co,
             g0_v, g1_v, g2_v, dfc_v, spt_v, dpt_v,
             b0, b1, b2, c0, c1, c2, sem):
    cid = lax.axis_index("c")
    sid = lax.axis_index("s")
    wid = sid * NC + cid

    pltpu.sync_copy(g0_hbm.at[wid], g0_v)
    pltpu.sync_copy(g1_hbm.at[wid], g1_v)
    pltpu.sync_copy(g2_hbm.at[wid], g2_v)
    pltpu.sync_copy(dfc_hbm.at[wid], dfc_v)
    pltpu.sync_copy(spt_hbm.at[wid], spt_v)
    pltpu.sync_copy(dpt_hbm.at[wid], dpt_v)

    def face_chunk(j, carry):
        cp0 = pltpu.async_copy(fe_hbm.at[g0_v.at[j]], b0, sem)
        cp1 = pltpu.async_copy(fe_hbm.at[g1_v.at[j]], b1, sem)
        cp2 = pltpu.async_copy(fe_hbm.at[g2_v.at[j]], b2, sem)
        cc0 = pltpu.async_copy(co_hbm.at[g0_v.at[j]], c0, sem)
        cc1 = pltpu.async_copy(co_hbm.at[g1_v.at[j]], c1, sem)
        cc2 = pltpu.async_copy(co_hbm.at[g2_v.at[j]], c2, sem)
        cp0.wait(); cp1.wait(); cp2.wait()
        cc0.wait(); cc1.wait(); cc2.wait()

        def mean_row(r, _):
            for cc in range(D // 16):
                sl = pl.ds(cc * 16, 16)
                b0[r, sl] = (b0[r, sl] + b1[r, sl] + b2[r, sl]) * THIRD
            c0[r, :] = (c0[r, :] + c1[r, :] + c2[r, :]) * THIRD
            return 0

        lax.fori_loop(0, CHUNK, mean_row, 0)
        sf = pltpu.async_copy(b0, out_fe.at[dfc_v.at[j]], sem)
        sc = pltpu.async_copy(c0, out_co.at[dfc_v.at[j]], sem)
        sf.wait(); sc.wait()
        return carry

    lax.fori_loop(0, NCHUNK, face_chunk, 0)

    def pt_chunk(j, carry):
        cpf = pltpu.async_copy(fe_hbm.at[spt_v.at[j]], b0, sem)
        cpc = pltpu.async_copy(co_hbm.at[spt_v.at[j]], c0, sem)
        cpf.wait(); cpc.wait()
        sf = pltpu.async_copy(b0, out_fe.at[dpt_v.at[j]], sem)
        sc = pltpu.async_copy(c0, out_co.at[dpt_v.at[j]], sem)
        sf.wait(); sc.wait()
        return carry

    lax.fori_loop(0, NCHUNK, pt_chunk, 0)


def kernel(coords, point_fe, point_batch, face_ds, face_batch):
    B = 8
    n_pts = point_batch.shape[0]
    n_fcs = face_batch.shape[0]
    total = n_pts + n_fcs
    npad = NW * NCHUNK * CHUNK

    ids = jnp.arange(B + 1, dtype=jnp.int32)
    pt_cum = jnp.searchsorted(point_batch, ids, side="left").astype(jnp.int32)
    fc_cum = jnp.searchsorted(face_batch, ids, side="left").astype(jnp.int32)
    pt_counts = pt_cum[1:] - pt_cum[:-1]
    cap = jnp.maximum(pt_counts - 1, 0)
    local = jnp.clip(face_ds, 0, cap[face_batch][:, None])
    g = pt_cum[face_batch][:, None] + local
    dest_fc = jnp.arange(n_fcs, dtype=jnp.int32) + pt_cum[face_batch + 1]
    dest_pt = jnp.arange(n_pts, dtype=jnp.int32) + fc_cum[point_batch]

    pad_src = jnp.arange(npad - n_fcs, dtype=jnp.int32)
    pad_dst = dest_pt[: npad - n_fcs]
    shape = (NW, NCHUNK, CHUNK)
    g0 = jnp.concatenate([g[:, 0], pad_src]).reshape(shape)
    g1 = jnp.concatenate([g[:, 1], pad_src]).reshape(shape)
    g2 = jnp.concatenate([g[:, 2], pad_src]).reshape(shape)
    dfc = jnp.concatenate([dest_fc, pad_dst]).reshape(shape)
    spt = jnp.concatenate(
        [jnp.arange(n_pts, dtype=jnp.int32), pad_src]).reshape(shape)
    dpt = jnp.concatenate([dest_pt, pad_dst]).reshape(shape)

    co_p = jnp.pad(coords, ((0, 0), (0, DC - coords.shape[1])))

    run = pl.kernel(
        _sc_body,
        out_type=(
            jax.ShapeDtypeStruct((total, D), jnp.float32),
            jax.ShapeDtypeStruct((total, DC), jnp.float32),
        ),
        mesh=plsc.VectorSubcoreMesh(
            core_axis_name="c", subcore_axis_name="s",
            num_cores=NC, num_subcores=NS),
        scratch_types=[
            pltpu.VMEM((NCHUNK, CHUNK), jnp.int32),
            pltpu.VMEM((NCHUNK, CHUNK), jnp.int32),
            pltpu.VMEM((NCHUNK, CHUNK), jnp.int32),
            pltpu.VMEM((NCHUNK, CHUNK), jnp.int32),
            pltpu.VMEM((NCHUNK, CHUNK), jnp.int32),
            pltpu.VMEM((NCHUNK, CHUNK), jnp.int32),
            pltpu.VMEM((CHUNK, D), jnp.float32),
            pltpu.VMEM((CHUNK, D), jnp.float32),
            pltpu.VMEM((CHUNK, D), jnp.float32),
            pltpu.VMEM((CHUNK, DC), jnp.float32),
            pltpu.VMEM((CHUNK, DC), jnp.float32),
            pltpu.VMEM((CHUNK, DC), jnp.float32),
            pltpu.SemaphoreType.DMA,
        ],
        compiler_params=pltpu.CompilerParams(use_tc_tiling_on_sc=False),
    )
    out_fe, out_co = run(point_fe, co_p, g0, g1, g2, dfc, spt, dpt)
    return out_co[:, : coords.shape[1]], out_fe


# speedup vs baseline: 14.9076x; 1.2876x over previous
import functools

import jax
import jax.numpy as jnp
import numpy as np
from jax import lax
from jax.experimental import pallas as pl
from jax.experimental.pallas import tpu as pltpu
from jax.experimental.pallas import tpu_sc as plsc

NC = 2
NS = 16
NW = NC * NS
CHUNK = 64
NCHUNK = 50
PER_W = NCHUNK * CHUNK
D = 128
DC = 16
THIRD = np.float32(1.0) / np.float32(3.0)


def _sc_body(fe_hbm, co_hbm, g0_hbm, g1_hbm, g2_hbm, dfc_hbm, spt_hbm,
             dpt_hbm, out_fe, out_co,
             g0_v, g1_v, g2_v, dfc_v, spt_v, dpt_v,
             fb00, fb01, fb02, fo0, fb10, fb11, fb12, fo1,
             fc00, fc01, fc02, foc0, fc10, fc11, fc12, foc1,
             pb0, pb1, pb2, pb3, pc0, pc1, pc2, pc3,
             semfg0, semfg1, semfs0, semfs1,
             sempg0, sempg1, sempg2, sempg3,
             semps0, semps1, semps2, semps3):
    FB = [[fb00, fb01, fb02], [fb10, fb11, fb12]]
    FO = [fo0, fo1]
    FC = [[fc00, fc01, fc02], [fc10, fc11, fc12]]
    FOC = [foc0, foc1]
    PB = [pb0, pb1, pb2, pb3]
    PC = [pc0, pc1, pc2, pc3]
    GV = [g0_v, g1_v, g2_v]
    SEMFG = [semfg0, semfg1]
    SEMFS = [semfs0, semfs1]
    SEMPG = [sempg0, sempg1, sempg2, sempg3]
    SEMPS = [semps0, semps1, semps2, semps3]

    cid = lax.axis_index("c")
    sid = lax.axis_index("s")
    wid = sid * NC + cid

    pltpu.sync_copy(g0_hbm.at[wid], g0_v)
    pltpu.sync_copy(g1_hbm.at[wid], g1_v)
    pltpu.sync_copy(g2_hbm.at[wid], g2_v)
    pltpu.sync_copy(dfc_hbm.at[wid], dfc_v)
    pltpu.sync_copy(spt_hbm.at[wid], spt_v)
    pltpu.sync_copy(dpt_hbm.at[wid], dpt_v)

    def face_g(j, sl, fn):
        for t in range(3):
            fn(fe_hbm.at[GV[t].at[j]], FB[sl][t], SEMFG[sl])
            fn(co_hbm.at[GV[t].at[j]], FC[sl][t], SEMFG[sl])

    def face_s(j, sl, fn):
        fn(FO[sl], out_fe.at[dfc_v.at[j]], SEMFS[sl])
        fn(FOC[sl], out_co.at[dfc_v.at[j]], SEMFS[sl])

    def pt_g(j, p, fn):
        fn(fe_hbm.at[spt_v.at[j]], PB[p], SEMPG[p])
        fn(co_hbm.at[spt_v.at[j]], PC[p], SEMPG[p])

    def pt_s(j, p, fn):
        fn(PB[p], out_fe.at[dpt_v.at[j]], SEMPS[p])
        fn(PC[p], out_co.at[dpt_v.at[j]], SEMPS[p])

    issue = pltpu.async_copy

    def drain(src, dst, sem):
        pltpu.make_async_copy(src, dst, sem).wait()

    def compute(sl):
        b0, b1, b2 = FB[sl]
        c0, c1, c2 = FC[sl]
        o, oc = FO[sl], FOC[sl]

        def mean_row(r, _):
            for g in range(D // 16):
                s = pl.ds(g * 16, 16)
                o[r, s] = (b0[r, s] + b1[r, s] + b2[r, s]) * THIRD
            oc[r, :] = (c0[r, :] + c1[r, :] + c2[r, :]) * THIRD
            return 0

        lax.fori_loop(0, CHUNK, mean_row, 0)

    def step(j, sl, p, p2, *, wait2=True, ahead=True):
        if wait2:
            pt_s(j - 2, p2, drain)
        if ahead:
            pt_g(j + 2, p2, issue)
        face_g(j, sl, drain)
        if wait2:
            face_s(j - 2, sl, drain)
        compute(sl)
        face_s(j, sl, issue)
        if ahead:
            face_g(j + 2, sl, issue)
        pt_g(j, p, drain)
        pt_s(j, p, issue)

    face_g(0, 0, issue)
    face_g(1, 1, issue)
    pt_g(0, 2, issue)
    pt_g(1, 3, issue)
    step(0, 0, 2, 0, wait2=False)
    step(1, 1, 3, 1, wait2=False)

    def outer(k, carry):
        jb = 4 * k + 2
        for s4 in range(4):
            step(jb + s4, s4 % 2, s4, (s4 + 2) % 4)
        return carry

    lax.fori_loop(0, (NCHUNK - 6) // 4, outer, 0)

    step(46, 0, 0, 2)
    step(47, 1, 1, 3)
    step(48, 0, 2, 0, ahead=False)
    step(49, 1, 3, 1, ahead=False)
    face_s(48, 0, drain)
    face_s(49, 1, drain)
    pt_s(48, 2, drain)
    pt_s(49, 3, drain)


def kernel(coords, point_fe, point_batch, face_ds, face_batch):
    B = 8
    n_pts = point_batch.shape[0]
    n_fcs = face_batch.shape[0]
    total = n_pts + n_fcs
    npad = NW * PER_W

    ids = jnp.arange(B + 1, dtype=jnp.int32)
    pt_cum = jnp.searchsorted(point_batch, ids, side="left").astype(jnp.int32)
    fc_cum = jnp.searchsorted(face_batch, ids, side="left").astype(jnp.int32)
    pt_counts = pt_cum[1:] - pt_cum[:-1]
    cap = jnp.maximum(pt_counts - 1, 0)
    local = jnp.clip(face_ds, 0, cap[face_batch][:, None])
    g = pt_cum[face_batch][:, None] + local
    dest_fc = jnp.arange(n_fcs, dtype=jnp.int32) + pt_cum[face_batch + 1]
    dest_pt = jnp.arange(n_pts, dtype=jnp.int32) + fc_cum[point_batch]

    pad_src = jnp.arange(npad - n_fcs, dtype=jnp.int32)
    pad_dst = dest_pt[: npad - n_fcs]
    shape = (NW, NCHUNK, CHUNK)
    g0 = jnp.concatenate([g[:, 0], pad_src]).reshape(shape)
    g1 = jnp.concatenate([g[:, 1], pad_src]).reshape(shape)
    g2 = jnp.concatenate([g[:, 2], pad_src]).reshape(shape)
    dfc = jnp.concatenate([dest_fc, pad_dst]).reshape(shape)
    spt = jnp.concatenate(
        [jnp.arange(n_pts, dtype=jnp.int32), pad_src]).reshape(shape)
    dpt = jnp.concatenate([dest_pt, pad_dst]).reshape(shape)

    co_p = jnp.pad(coords, ((0, 0), (0, DC - coords.shape[1])))

    fe_buf = pltpu.VMEM((CHUNK, D), jnp.float32)
    co_buf = pltpu.VMEM((CHUNK, DC), jnp.float32)
    idx_buf = pltpu.VMEM((NCHUNK, CHUNK), jnp.int32)
    run = pl.kernel(
        _sc_body,
        out_type=(
            jax.ShapeDtypeStruct((total, D), jnp.float32),
            jax.ShapeDtypeStruct((total, DC), jnp.float32),
        ),
        mesh=plsc.VectorSubcoreMesh(
            core_axis_name="c", subcore_axis_name="s",
            num_cores=NC, num_subcores=NS),
        scratch_types=(
            [idx_buf] * 6
            + [fe_buf] * 8
            + [co_buf] * 8
            + [fe_buf] * 4
            + [co_buf] * 4
            + [pltpu.SemaphoreType.DMA] * 12
        ),
        compiler_params=pltpu.CompilerParams(use_tc_tiling_on_sc=False),
    )
    out_fe, out_co = run(point_fe, co_p, g0, g1, g2, dfc, spt, dpt)
    return out_co[:, : coords.shape[1]], out_fe


# speedup vs baseline: 15.6644x; 1.0508x over previous
import functools

import jax
import jax.numpy as jnp
import numpy as np
from jax import lax
from jax.experimental import pallas as pl
from jax.experimental.pallas import tpu as pltpu
from jax.experimental.pallas import tpu_sc as plsc

NC = 2
NS = 16
NW = NC * NS
CHUNK = 64
NCHUNK = 50
PER_W = NCHUNK * CHUNK
D = 128
DC = 16
THIRD = np.float32(1.0) / np.float32(3.0)


def _sc_body(fe_hbm, co_hbm, g0_hbm, g1_hbm, g2_hbm, dfc_hbm, spt_hbm,
             dpt_hbm, out_fe, out_co,
             g0_v, g1_v, g2_v, dfc_v, spt_v, dpt_v,
             fb00, fb01, fb02, fo0, fb10, fb11, fb12, fo1,
             fc00, fc01, fc02, foc0, fc10, fc11, fc12, foc1,
             pb0, pb1, pb2, pb3, pc0, pc1, pc2, pc3,
             semfg0, semfg1, semfs0, semfs1,
             sempg0, sempg1, sempg2, sempg3,
             semps0, semps1, semps2, semps3):
    FB = [[fb00, fb01, fb02], [fb10, fb11, fb12]]
    FO = [fo0, fo1]
    FC = [[fc00, fc01, fc02], [fc10, fc11, fc12]]
    FOC = [foc0, foc1]
    PB = [pb0, pb1, pb2, pb3]
    PC = [pc0, pc1, pc2, pc3]
    GV = [g0_v, g1_v, g2_v]
    SEMFG = [semfg0, semfg1]
    SEMFS = [semfs0, semfs1]
    SEMPG = [sempg0, sempg1, sempg2, sempg3]
    SEMPS = [semps0, semps1, semps2, semps3]

    cid = lax.axis_index("c")
    sid = lax.axis_index("s")
    wid = sid * NC + cid

    pltpu.sync_copy(g0_hbm.at[wid], g0_v)
    pltpu.sync_copy(g1_hbm.at[wid], g1_v)
    pltpu.sync_copy(g2_hbm.at[wid], g2_v)
    pltpu.sync_copy(dfc_hbm.at[wid], dfc_v)
    pltpu.sync_copy(spt_hbm.at[wid], spt_v)
    pltpu.sync_copy(dpt_hbm.at[wid], dpt_v)

    def face_g(j, sl, fn):
        for t in range(3):
            fn(fe_hbm.at[GV[t].at[j]], FB[sl][t], SEMFG[sl])
            fn(co_hbm.at[GV[t].at[j]], FC[sl][t], SEMFG[sl])

    def face_s(j, sl, fn):
        fn(FO[sl], out_fe.at[dfc_v.at[j]], SEMFS[sl])
        fn(FOC[sl], out_co.at[dfc_v.at[j]], SEMFS[sl])

    def pt_g(j, p, fn):
        fn(fe_hbm.at[spt_v.at[j]], PB[p], SEMPG[p])
        fn(co_hbm.at[spt_v.at[j]], PC[p], SEMPG[p])

    def pt_s(j, p, fn):
        fn(PB[p], out_fe.at[dpt_v.at[j]], SEMPS[p])
        fn(PC[p], out_co.at[dpt_v.at[j]], SEMPS[p])

    issue = pltpu.async_copy

    def drain(src, dst, sem):
        pltpu.make_async_copy(src, dst, sem).wait()

    def compute(sl):
        b0, b1, b2 = FB[sl]
        c0, c1, c2 = FC[sl]
        o, oc = FO[sl], FOC[sl]

        def mean_row(r, _):
            for g in range(D // 16):
                s = pl.ds(g * 16, 16)
                o[r, s] = (b0[r, s] + b1[r, s] + b2[r, s]) * THIRD
            oc[r, :] = (c0[r, :] + c1[r, :] + c2[r, :]) * THIRD
            return 0

        lax.fori_loop(0, CHUNK, mean_row, 0)

    def step(j, sl, p, p2, *, wait2=True, ahead=True):
        if wait2:
            pt_s(j - 2, p2, drain)
        if ahead:
            pt_g(j + 2, p2, issue)
        face_g(j, sl, drain)
        if wait2:
            face_s(j - 2, sl, drain)
        compute(sl)
        face_s(j, sl, issue)
        if ahead:
            face_g(j + 2, sl, issue)
        pt_g(j, p, drain)
        pt_s(j, p, issue)

    face_g(0, 0, issue)
    face_g(1, 1, issue)
    pt_g(0, 2, issue)
    pt_g(1, 3, issue)
    step(0, 0, 2, 0, wait2=False)
    step(1, 1, 3, 1, wait2=False)

    def outer(k, carry):
        jb = 4 * k + 2
        for s4 in range(4):
            step(jb + s4, s4 % 2, s4, (s4 + 2) % 4)
        return carry

    lax.fori_loop(0, (NCHUNK - 6) // 4, outer, 0)

    step(46, 0, 0, 2)
    step(47, 1, 1, 3)
    step(48, 0, 2, 0, ahead=False)
    step(49, 1, 3, 1, ahead=False)
    face_s(48, 0, drain)
    face_s(49, 1, drain)
    pt_s(48, 2, drain)
    pt_s(49, 3, drain)


def kernel(coords, point_fe, point_batch, face_ds, face_batch):
    B = 8
    n_pts = point_batch.shape[0]
    n_fcs = face_batch.shape[0]
    total = n_pts + n_fcs
    npad = NW * PER_W

    ids8 = jnp.arange(B, dtype=jnp.int32)
    pt_counts = jnp.sum(
        (point_batch[:, None] == ids8[None, :]).astype(jnp.int32), axis=0)
    fc_counts = jnp.sum(
        (face_batch[:, None] == ids8[None, :]).astype(jnp.int32), axis=0)
    zero = jnp.zeros((1,), dtype=jnp.int32)
    pt_cum = jnp.concatenate([zero, jnp.cumsum(pt_counts)])
    fc_cum = jnp.concatenate([zero, jnp.cumsum(fc_counts)])
    cap = jnp.maximum(pt_counts - 1, 0)
    local = jnp.clip(face_ds, 0, cap[face_batch][:, None])
    g = pt_cum[face_batch][:, None] + local
    dest_fc = jnp.arange(n_fcs, dtype=jnp.int32) + pt_cum[face_batch + 1]
    dest_pt = jnp.arange(n_pts, dtype=jnp.int32) + fc_cum[point_batch]

    pad_src = jnp.arange(npad - n_fcs, dtype=jnp.int32)
    pad_dst = dest_pt[: npad - n_fcs]
    shape = (NW, NCHUNK, CHUNK)
    g0 = jnp.concatenate([g[:, 0], pad_src]).reshape(shape)
    g1 = jnp.concatenate([g[:, 1], pad_src]).reshape(shape)
    g2 = jnp.concatenate([g[:, 2], pad_src]).reshape(shape)
    dfc = jnp.concatenate([dest_fc, pad_dst]).reshape(shape)
    spt = jnp.concatenate(
        [jnp.arange(n_pts, dtype=jnp.int32), pad_src]).reshape(shape)
    dpt = jnp.concatenate([dest_pt, pad_dst]).reshape(shape)

    co_p = jnp.pad(coords, ((0, 0), (0, DC - coords.shape[1])))

    fe_buf = pltpu.VMEM((CHUNK, D), jnp.float32)
    co_buf = pltpu.VMEM((CHUNK, DC), jnp.float32)
    idx_buf = pltpu.VMEM((NCHUNK, CHUNK), jnp.int32)
    run = pl.kernel(
        _sc_body,
        out_type=(
            jax.ShapeDtypeStruct((total, D), jnp.float32),
            jax.ShapeDtypeStruct((total, DC), jnp.float32),
        ),
        mesh=plsc.VectorSubcoreMesh(
            core_axis_name="c", subcore_axis_name="s",
            num_cores=NC, num_subcores=NS),
        scratch_types=(
            [idx_buf] * 6
            + [fe_buf] * 8
            + [co_buf] * 8
            + [fe_buf] * 4
            + [co_buf] * 4
            + [pltpu.SemaphoreType.DMA] * 12
        ),
        compiler_params=pltpu.CompilerParams(use_tc_tiling_on_sc=False),
    )
    out_fe, out_co = run(point_fe, co_p, g0, g1, g2, dfc, spt, dpt)
    return out_co[:, : coords.shape[1]], out_fe


# speedup vs baseline: 20.2148x; 1.2905x over previous
import functools

import jax
import jax.numpy as jnp
import numpy as np
from jax import lax
from jax.experimental import pallas as pl
from jax.experimental.pallas import tpu as pltpu
from jax.experimental.pallas import tpu_sc as plsc

NC = 2
NS = 16
NW = NC * NS
CHUNK = 100
NCHUNK = 32
PER_W = NCHUNK * CHUNK
D = 128
DC = 16
THIRD = np.float32(1.0) / np.float32(3.0)


def _body(fe_hbm, g0_hbm, g1_hbm, g2_hbm, dfc_hbm, spt_hbm, dpt_hbm, out_fe,
          g0_v, g1_v, g2_v, dfc_v, spt_v, dpt_v,
          b00, b01, b02, o0, b10, b11, b12, o1,
          semfg0, semfg1, semfs0, semfs1,
          sempg0, sempg1, sempg2, sempg3,
          semps0, semps1, semps2, semps3):
    width = fe_hbm.shape[1]
    FB = [[b00, b01, b02], [b10, b11, b12]]
    FO = [o0, o1]
    PB = [b00, b01, b02, o0]
    GV = [g0_v, g1_v, g2_v]
    SEMFG = [semfg0, semfg1]
    SEMFS = [semfs0, semfs1]
    SEMPG = [sempg0, sempg1, sempg2, sempg3]
    SEMPS = [semps0, semps1, semps2, semps3]

    cid = lax.axis_index("c")
    sid = lax.axis_index("s")
    wid = sid * NC + cid

    pltpu.sync_copy(g0_hbm.at[wid], g0_v)
    pltpu.sync_copy(g1_hbm.at[wid], g1_v)
    pltpu.sync_copy(g2_hbm.at[wid], g2_v)
    pltpu.sync_copy(dfc_hbm.at[wid], dfc_v)
    pltpu.sync_copy(spt_hbm.at[wid], spt_v)
    pltpu.sync_copy(dpt_hbm.at[wid], dpt_v)

    issue = pltpu.async_copy

    def drain(src, dst, sem):
        pltpu.make_async_copy(src, dst, sem).wait()

    def face_g(j, sl, fn):
        for t in range(3):
            fn(fe_hbm.at[GV[t].at[j]], FB[sl][t], SEMFG[sl])

    def face_s(j, sl, fn):
        fn(FO[sl], out_fe.at[dfc_v.at[j]], SEMFS[sl])

    def pt_g(j, p, fn):
        fn(fe_hbm.at[spt_v.at[j]], PB[p], SEMPG[p])

    def pt_s(j, p, fn):
        fn(PB[p], out_fe.at[dpt_v.at[j]], SEMPS[p])

    def compute(sl):
        b0, b1, b2 = FB[sl]
        o = FO[sl]

        def mean_row(r, _):
            for g in range(width // 16):
                s = pl.ds(g * 16, 16)
                o[r, s] = (b0[r, s] + b1[r, s] + b2[r, s]) * THIRD
            return 0

        lax.fori_loop(0, CHUNK, mean_row, 0)

    def fstep(j, sl, *, wait2=True, ahead=True):
        face_g(j, sl, drain)
        if wait2:
            face_s(j - 2, sl, drain)
        compute(sl)
        face_s(j, sl, issue)
        if ahead:
            face_g(j + 2, sl, issue)

    face_g(0, 0, issue)
    face_g(1, 1, issue)
    fstep(0, 0, wait2=False)
    fstep(1, 1, wait2=False)

    def fouter(k, carry):
        jb = 2 * k + 2
        fstep(jb, 0)
        fstep(jb + 1, 1)
        return carry

    lax.fori_loop(0, (NCHUNK - 4) // 2, fouter, 0)
    fstep(NCHUNK - 2, 0, ahead=False)
    fstep(NCHUNK - 1, 1, ahead=False)
    face_s(NCHUNK - 2, 0, drain)
    face_s(NCHUNK - 1, 1, drain)

    def pstep(j, p, p2, *, wait2=True, ahead=True):
        if wait2:
            pt_s(j - 2, p2, drain)
        if ahead:
            pt_g(j + 2, p2, issue)
        pt_g(j, p, drain)
        pt_s(j, p, issue)

    pt_g(0, 0, issue)
    pt_g(1, 1, issue)
    pstep(0, 0, 2, wait2=False)
    pstep(1, 1, 3, wait2=False)

    def pouter(k, carry):
        jb = 4 * k + 2
        for s4 in range(4):
            pstep(jb + s4, (2 + s4) % 4, s4 % 4)
        return carry

    lax.fori_loop(0, (NCHUNK - 4) // 4, pouter, 0)
    pstep(NCHUNK - 2, 2, 0, ahead=False)
    pstep(NCHUNK - 1, 3, 1, ahead=False)
    pt_s(NCHUNK - 2, 2, drain)
    pt_s(NCHUNK - 1, 3, drain)


def _make_kernel(n_rows, total, width, tc_tiling):
    buf = pltpu.VMEM((CHUNK, width), jnp.float32)
    idx_buf = pltpu.VMEM((NCHUNK, CHUNK), jnp.int32)
    return pl.kernel(
        _body,
        out_type=jax.ShapeDtypeStruct((total, width), jnp.float32),
        mesh=plsc.VectorSubcoreMesh(
            core_axis_name="c", subcore_axis_name="s",
            num_cores=NC, num_subcores=NS),
        scratch_types=(
            [idx_buf] * 6 + [buf] * 8 + [pltpu.SemaphoreType.DMA] * 12
        ),
        compiler_params=pltpu.CompilerParams(use_tc_tiling_on_sc=tc_tiling),
    )


def kernel(coords, point_fe, point_batch, face_ds, face_batch):
    B = 8
    n_pts = point_batch.shape[0]
    n_fcs = face_batch.shape[0]
    total = n_pts + n_fcs
    npad = NW * PER_W

    ids8 = jnp.arange(B, dtype=jnp.int32)
    pt_counts = jnp.sum(
        (point_batch[:, None] == ids8[None, :]).astype(jnp.int32), axis=0)
    fc_counts = jnp.sum(
        (face_batch[:, None] == ids8[None, :]).astype(jnp.int32), axis=0)
    zero = jnp.zeros((1,), dtype=jnp.int32)
    pt_cum = jnp.concatenate([zero, jnp.cumsum(pt_counts)])
    fc_cum = jnp.concatenate([zero, jnp.cumsum(fc_counts)])
    cap = jnp.maximum(pt_counts - 1, 0)
    local = jnp.clip(face_ds, 0, cap[face_batch][:, None])
    g = pt_cum[face_batch][:, None] + local
    dest_fc = jnp.arange(n_fcs, dtype=jnp.int32) + pt_cum[face_batch + 1]
    dest_pt = jnp.arange(n_pts, dtype=jnp.int32) + fc_cum[point_batch]

    pad_src = jnp.arange(npad - n_fcs, dtype=jnp.int32)
    pad_dst = dest_pt[: npad - n_fcs]
    shape = (NW, NCHUNK, CHUNK)
    g0 = jnp.concatenate([g[:, 0], pad_src]).reshape(shape)
    g1 = jnp.concatenate([g[:, 1], pad_src]).reshape(shape)
    g2 = jnp.concatenate([g[:, 2], pad_src]).reshape(shape)
    dfc = jnp.concatenate([dest_fc, pad_dst]).reshape(shape)
    spt = jnp.concatenate(
        [jnp.arange(n_pts, dtype=jnp.int32), pad_src]).reshape(shape)
    dpt = jnp.concatenate([dest_pt, pad_dst]).reshape(shape)

    co_p = jnp.pad(coords, ((0, 0), (0, DC - coords.shape[1])))

    run_co = _make_kernel(n_pts, total, DC, False)
    run_fe = _make_kernel(n_pts, total, D, True)
    out_co = run_co(co_p, g0, g1, g2, dfc, spt, dpt)
    out_fe = run_fe(point_fe, g0, g1, g2, dfc, spt, dpt)
    return out_co[:, : coords.shape[1]], out_fe
